# baseline (speedup 1.0000x reference)
import jax
import jax.numpy as jnp
from jax import lax
from jax.experimental import pallas as pl

HIDDEN = 4096
N_EXPERTS = 64
TOP_K = 8
ROW_BLOCK = 512
COL_CHUNK = 128


def _router_body(x_ref, w_ref, idx_ref, wt_ref, p_ref):
    x_blk = x_ref[...]
    w = w_ref[...]
    logits_t = lax.dot_general(
        w, x_blk, (((1,), (1,)), ((), ())),
        preferred_element_type=jnp.float32)

    for c in range(ROW_BLOCK // COL_CHUNK):
        sl = slice(c * COL_CHUNK, (c + 1) * COL_CHUNK)
        lt = logits_t[:, sl]
        m = jnp.max(lt, axis=0, keepdims=True)
        e = jnp.exp(lt - m)
        probs_t = e / jnp.sum(e, axis=0, keepdims=True)
        p_ref[sl, :] = probs_t.T

        iota = lax.broadcasted_iota(jnp.int32, probs_t.shape, 0)
        vals = probs_t
        wt_rows = []
        idx_rows = []
        for _ in range(TOP_K):
            mx = jnp.max(vals, axis=0, keepdims=True)
            cand = jnp.where(vals == mx, iota, N_EXPERTS)
            amin = jnp.min(cand, axis=0, keepdims=True)
            wt_rows.append(mx)
            idx_rows.append(amin)
            vals = jnp.where(iota == amin, -jnp.inf, vals)

        weights_t = jnp.concatenate(wt_rows, axis=0)
        weights_t = weights_t / (
            jnp.sum(weights_t, axis=0, keepdims=True) + 1e-9)
        idx_ref[sl, :] = jnp.concatenate(idx_rows, axis=0).T
        wt_ref[sl, :] = weights_t.T


@jax.jit
def _router(flat, w):
    n_rows = flat.shape[0]
    return pl.pallas_call(
        _router_body,
        grid=(n_rows // ROW_BLOCK,),
        in_specs=[
            pl.BlockSpec((ROW_BLOCK, HIDDEN), lambda i: (i, 0)),
            pl.BlockSpec((N_EXPERTS, HIDDEN), lambda i: (0, 0)),
        ],
        out_specs=[
            pl.BlockSpec((ROW_BLOCK, TOP_K), lambda i: (i, 0)),
            pl.BlockSpec((ROW_BLOCK, TOP_K), lambda i: (i, 0)),
            pl.BlockSpec((ROW_BLOCK, N_EXPERTS), lambda i: (i, 0)),
        ],
        out_shape=[
            jax.ShapeDtypeStruct((n_rows, TOP_K), jnp.int32),
            jax.ShapeDtypeStruct((n_rows, TOP_K), jnp.float32),
            jax.ShapeDtypeStruct((n_rows, N_EXPERTS), jnp.float32),
        ],
    )(flat, w)


def kernel(x, W):
    flat = x.reshape(-1, x.shape[-1])
    indices, weights, probs = _router(flat, W)
    return indices, weights.astype(x.dtype), probs


# speedup vs baseline: 1.0539x; 1.0539x over previous
import jax
import jax.numpy as jnp
from jax import lax
from jax.experimental import pallas as pl

HIDDEN = 4096
N_EXPERTS = 64
TOP_K = 8
ROW_BLOCK = 1024
COL_CHUNK = 256


def _router_body(x_ref, w_ref, idx_ref, wt_ref, p_ref):
    x_blk = x_ref[...]
    w = w_ref[...]
    logits_t = lax.dot_general(
        w, x_blk, (((1,), (1,)), ((), ())),
        preferred_element_type=jnp.float32)

    for c in range(ROW_BLOCK // COL_CHUNK):
        sl = slice(c * COL_CHUNK, (c + 1) * COL_CHUNK)
        lt = logits_t[:, sl]
        m = jnp.max(lt, axis=0, keepdims=True)
        e = jnp.exp(lt - m)
        probs_t = e / jnp.sum(e, axis=0, keepdims=True)
        p_ref[sl, :] = probs_t.T

        iota = lax.broadcasted_iota(jnp.int32, probs_t.shape, 0)
        vals = probs_t
        wt_rows = []
        idx_rows = []
        for _ in range(TOP_K):
            mx = jnp.max(vals, axis=0, keepdims=True)
            cand = jnp.where(vals == mx, iota, N_EXPERTS)
            amin = jnp.min(cand, axis=0, keepdims=True)
            wt_rows.append(mx)
            idx_rows.append(amin)
            vals = jnp.where(iota == amin, -jnp.inf, vals)

        weights_t = jnp.concatenate(wt_rows, axis=0)
        weights_t = weights_t / (
            jnp.sum(weights_t, axis=0, keepdims=True) + 1e-9)
        idx_ref[sl, :] = jnp.concatenate(idx_rows, axis=0).T
        wt_ref[sl, :] = weights_t.T


@jax.jit
def _router(flat, w):
    n_rows = flat.shape[0]
    return pl.pallas_call(
        _router_body,
        grid=(n_rows // ROW_BLOCK,),
        in_specs=[
            pl.BlockSpec((ROW_BLOCK, HIDDEN), lambda i: (i, 0)),
            pl.BlockSpec((N_EXPERTS, HIDDEN), lambda i: (0, 0)),
        ],
        out_specs=[
            pl.BlockSpec((ROW_BLOCK, TOP_K), lambda i: (i, 0)),
            pl.BlockSpec((ROW_BLOCK, TOP_K), lambda i: (i, 0)),
            pl.BlockSpec((ROW_BLOCK, N_EXPERTS), lambda i: (i, 0)),
        ],
        out_shape=[
            jax.ShapeDtypeStruct((n_rows, TOP_K), jnp.int32),
            jax.ShapeDtypeStruct((n_rows, TOP_K), jnp.float32),
            jax.ShapeDtypeStruct((n_rows, N_EXPERTS), jnp.float32),
        ],
    )(flat, w)


def kernel(x, W):
    flat = x.reshape(-1, x.shape[-1])
    indices, weights, probs = _router(flat, W)
    return indices, weights.astype(x.dtype), probs
